# baseline (speedup 1.0000x reference)
import functools

import jax
import jax.numpy as jnp
from jax import lax
from jax.experimental import pallas as pl
from jax.experimental.pallas import tpu as pltpu
from jax.experimental.pallas import tpu_sc as plsc

NC = 2
NS = 16
NW = NC * NS
IDXW = 128
GPG = 4
ROWS = IDXW * GPG
SCALE = 8.0


def _embed_sc(table, idx3, n_groups, b_per_w, D, N):
    mesh = plsc.VectorSubcoreMesh(core_axis_name="c", subcore_axis_name="s")

    @functools.partial(
        pl.kernel,
        mesh=mesh,
        compiler_params=pltpu.CompilerParams(use_tc_tiling_on_sc=False),
        out_type=jax.ShapeDtypeStruct((N, D), jnp.float32),
        scratch_types=[
            pltpu.VMEM((n_groups * GPG, IDXW), jnp.int32),
            pltpu.VMEM((ROWS, D), jnp.float32),
            pltpu.VMEM((ROWS, D), jnp.float32),
            pltpu.SemaphoreType.DMA,
            pltpu.SemaphoreType.DMA,
        ],
    )
    def k(table_hbm, idx_hbm, out_hbm, idx_v, buf0, buf1, sem0, sem1):
        wid = lax.axis_index("s") * NC + lax.axis_index("c")
        base = wid * b_per_w
        pltpu.sync_copy(idx_hbm.at[wid], idx_v)

        def fire(g, buf, sem):
            for j in range(GPG):
                pltpu.async_copy(
                    table_hbm.at[idx_v.at[g * GPG + j]],
                    buf.at[pl.ds(j * IDXW, IDXW)],
                    sem,
                )

        def drain(g, buf, sem):
            for j in range(GPG):
                pltpu.make_async_copy(
                    table_hbm.at[idx_v.at[g * GPG + j]],
                    buf.at[pl.ds(j * IDXW, IDXW)],
                    sem,
                ).wait()

        def scale_store(g, buf):
            def row(r, c):
                for cc in range(D // 16):
                    sl = pl.ds(cc * 16, 16)
                    buf[r, sl] = buf[r, sl] * SCALE
                return c

            lax.fori_loop(0, ROWS, row, 0)
            pltpu.sync_copy(buf, out_hbm.at[pl.ds(base + g * ROWS, ROWS)])

        fire(0, buf0, sem0)

        def body(g, c):
            for b in range(2):
                buf = (buf0, buf1)[b]
                sem = (sem0, sem1)[b]
                obuf = (buf0, buf1)[1 - b]
                osem = (sem0, sem1)[1 - b]

                @pl.when(lax.rem(g, 2) == b)
                def _():
                    @pl.when(g + 1 < n_groups)
                    def _():
                        fire(g + 1, obuf, osem)

                    drain(g, buf, sem)
                    scale_store(g, buf)

            return c

        lax.fori_loop(0, n_groups, body, 0)

    return k(table, idx3)


def kernel(X, embedding_matrix):
    B, H = X.shape
    V, D = embedding_matrix.shape
    N = B * H
    idx = X.reshape(N).astype(jnp.int32)
    b_per_w = N // NW
    n_groups = b_per_w // ROWS
    idx3 = idx.reshape(NW, n_groups * GPG, IDXW)
    out = _embed_sc(embedding_matrix, idx3, n_groups, b_per_w, D, N)
    return out.reshape(B, H, D)


# speedup vs baseline: 1.0122x; 1.0122x over previous
import functools

import jax
import jax.numpy as jnp
from jax import lax
from jax.experimental import pallas as pl
from jax.experimental.pallas import tpu as pltpu
from jax.experimental.pallas import tpu_sc as plsc

NC = 2
NS = 16
NW = NC * NS
GPG = 4
SCALE = 8.0


def _embed_sc(table, X):
    B, H = X.shape
    V, D = table.shape
    b_per_w = B // NW
    n_groups = b_per_w // GPG

    mesh = plsc.VectorSubcoreMesh(core_axis_name="c", subcore_axis_name="s")

    @functools.partial(
        pl.kernel,
        mesh=mesh,
        compiler_params=pltpu.CompilerParams(use_tc_tiling_on_sc=False),
        out_type=jax.ShapeDtypeStruct((B, H, D), jnp.float32),
        scratch_types=[
            pltpu.VMEM((b_per_w, H), jnp.int32),
            pltpu.VMEM((GPG, H, D), jnp.float32),
            pltpu.VMEM((GPG, H, D), jnp.float32),
            pltpu.SemaphoreType.DMA,
            pltpu.SemaphoreType.DMA,
        ],
    )
    def k(table_hbm, idx_hbm, out_hbm, idx_v, buf0, buf1, sem0, sem1):
        wid = lax.axis_index("s") * NC + lax.axis_index("c")
        wb = wid * b_per_w
        pltpu.sync_copy(idx_hbm.at[pl.ds(wb, b_per_w)], idx_v)

        def fire(g, buf, sem):
            for j in range(GPG):
                pltpu.async_copy(
                    table_hbm.at[idx_v.at[g * GPG + j]],
                    buf.at[j],
                    sem,
                )

        def drain(g, buf, sem):
            for j in range(GPG):
                pltpu.make_async_copy(
                    table_hbm.at[idx_v.at[g * GPG + j]],
                    buf.at[j],
                    sem,
                ).wait()

        def scale_store(g, buf):
            def row(r, c):
                for j in range(GPG):
                    for cc in range(D // 16):
                        sl = pl.ds(cc * 16, 16)
                        buf[j, r, sl] = buf[j, r, sl] * SCALE
                return c

            lax.fori_loop(0, H, row, 0)
            pltpu.sync_copy(buf, out_hbm.at[pl.ds(wb + g * GPG, GPG)])

        fire(0, buf0, sem0)

        def body(g, c):
            for b in range(2):
                buf = (buf0, buf1)[b]
                sem = (sem0, sem1)[b]
                obuf = (buf0, buf1)[1 - b]
                osem = (sem0, sem1)[1 - b]

                @pl.when(lax.rem(g, 2) == b)
                def _():
                    @pl.when(g + 1 < n_groups)
                    def _():
                        fire(g + 1, obuf, osem)

                    drain(g, buf, sem)
                    scale_store(g, buf)

            return c

        lax.fori_loop(0, n_groups, body, 0)

    return k(table, X)


def kernel(X, embedding_matrix):
    return _embed_sc(embedding_matrix, X)


# speedup vs baseline: 1.0658x; 1.0530x over previous
import functools

import jax
import jax.numpy as jnp
from jax import lax
from jax.experimental import pallas as pl
from jax.experimental.pallas import tpu as pltpu
from jax.experimental.pallas import tpu_sc as plsc

NC = 2
NS = 16
NW = NC * NS
GPG = 4
SCALE = 8.0


def _embed_sc(table, X):
    B, H = X.shape
    V2, D = table.shape
    b_per_w = B // NW
    n_groups = b_per_w // GPG

    mesh = plsc.VectorSubcoreMesh(core_axis_name="c", subcore_axis_name="s")

    @functools.partial(
        pl.kernel,
        mesh=mesh,
        compiler_params=pltpu.CompilerParams(use_tc_tiling_on_sc=False),
        out_type=jax.ShapeDtypeStruct((B, H, D), jnp.float32),
        scratch_types=[
            pltpu.VMEM((b_per_w, H), jnp.int32),
            pltpu.VMEM((GPG, H, D), jnp.float32),
            pltpu.VMEM((GPG, H, D), jnp.float32),
            pltpu.SemaphoreType.DMA,
            pltpu.SemaphoreType.DMA,
        ],
    )
    def k(table_hbm, idx_hbm, out_hbm, idx_v, buf0, buf1, sem0, sem1):
        wid = lax.axis_index("s") * NC + lax.axis_index("c")
        wb = wid * b_per_w
        pltpu.sync_copy(idx_hbm.at[pl.ds(wb, b_per_w)], idx_v)

        def fire(g, buf, sem):
            for j in range(GPG):
                pltpu.async_copy(
                    table_hbm.at[idx_v.at[g * GPG + j]],
                    buf.at[j],
                    sem,
                )

        def drain(g, buf, sem):
            for j in range(GPG):
                pltpu.make_async_copy(
                    table_hbm.at[idx_v.at[g * GPG + j]],
                    buf.at[j],
                    sem,
                ).wait()

        def scale_store(g, buf):
            def row(r, c):
                for j in range(GPG):
                    for cc in range(D // 16):
                        sl = pl.ds(cc * 16, 16)
                        buf[j, r, sl] = buf[j, r, sl] * SCALE
                return c

            lax.fori_loop(0, H, row, 0)
            pltpu.sync_copy(buf, out_hbm.at[pl.ds(wb + g * GPG, GPG)])

        fire(0, buf0, sem0)

        def body(g, c):
            for b in range(2):
                buf = (buf0, buf1)[b]
                sem = (sem0, sem1)[b]
                obuf = (buf0, buf1)[1 - b]
                osem = (sem0, sem1)[1 - b]

                @pl.when(lax.rem(g, 2) == b)
                def _():
                    @pl.when(g + 1 < n_groups)
                    def _():
                        fire(g + 1, obuf, osem)

                    drain(g, buf, sem)
                    scale_store(g, buf)

            return c

        lax.fori_loop(0, n_groups, body, 0)

    return k(table, X)


def kernel(X, embedding_matrix):
    V, D = embedding_matrix.shape
    t2 = jnp.pad(embedding_matrix, ((0, 0), (0, 128 - D))).reshape(2 * V, D)
    X2 = X.astype(jnp.int32) * 2
    return _embed_sc(t2, X2)
